# baseline (speedup 1.0000x reference)
import functools

import jax
import jax.numpy as jnp
from jax import lax
from jax.experimental import pallas as pl
from jax.experimental.pallas import tpu as pltpu
from jax.experimental.pallas import tpu_sc as plsc

EMBED_DIM = 32
BATCH = 16384
N_FIELDS = 26
BBLK = 128
NUM_WORKERS = 32
N_UNITS = N_FIELDS * (BATCH // BBLK)
UPW = N_UNITS // NUM_WORKERS


def _build_gather():
    mesh = plsc.VectorSubcoreMesh(core_axis_name="c", subcore_axis_name="s")
    nbb = BATCH // BBLK

    @functools.partial(
        pl.kernel,
        mesh=mesh,
        compiler_params=pltpu.CompilerParams(
            use_tc_tiling_on_sc=False, needs_layout_passes=False),
        out_type=jax.ShapeDtypeStruct(
            (N_FIELDS, EMBED_DIM // 8, nbb, 8, BBLK), jnp.float32),
        scratch_types=[
            pltpu.VMEM((UPW, BBLK), jnp.int32),
            pltpu.VMEM((BBLK, EMBED_DIM), jnp.float32),
            pltpu.VMEM((EMBED_DIM // 8, 8, BBLK), jnp.float32),
            pltpu.SemaphoreType.DMA,
            pltpu.SemaphoreType.DMA,
        ],
    )
    def gather_kernel(idx_hbm, table_hbm, out_hbm, idx_v, rows_v, w_v,
                      gsem, osem):
        wid = lax.axis_index("s") * 2 + lax.axis_index("c")
        ubase = wid * UPW
        pltpu.sync_copy(idx_hbm.at[pl.ds(ubase, UPW)], idx_v)
        iota16 = lax.iota(jnp.int32, 16)
        rvecs = [iota16 + 16 * k for k in range(BBLK // 16)]
        cvecs = [jnp.full((16,), d, jnp.int32) for d in range(EMBED_DIM)]

        def unit_body(g, _):
            c = ubase + g
            f = c // nbb
            bb = lax.rem(c, nbb)
            pltpu.async_copy(table_hbm.at[idx_v.at[g]], rows_v, gsem).wait()
            for d in range(EMBED_DIM):
                for k in range(BBLK // 16):
                    v = plsc.load_gather(rows_v, [rvecs[k], cvecs[d]])
                    w_v[d // 8, d % 8, pl.ds(16 * k, 16)] = v
            outs = []
            for d4 in range(EMBED_DIM // 8):
                outs.append(
                    pltpu.async_copy(
                        w_v.at[d4], out_hbm.at[f, d4, bb], osem))
            for cp in outs:
                cp.wait()
            return 0

        lax.fori_loop(0, UPW, unit_body, 0)

    return gather_kernel


def kernel(x, embeddings):
    batch, n_fields = x.shape
    idxT = x.T.astype(jnp.int32).reshape(N_UNITS, BBLK)
    out5 = _build_gather()(idxT, embeddings)
    return out5.transpose(2, 4, 0, 1, 3).reshape(batch, n_fields, EMBED_DIM)


# speedup vs baseline: 1.1740x; 1.1740x over previous
import functools

import jax
import jax.numpy as jnp
from jax import lax
from jax.experimental import pallas as pl
from jax.experimental.pallas import tpu as pltpu
from jax.experimental.pallas import tpu_sc as plsc

EMBED_DIM = 32
CHUNK = 1024
NUM_WORKERS = 32


def _build_gather(total_rows: int):
    n_chunks = total_rows // CHUNK
    cpw = n_chunks // NUM_WORKERS

    mesh = plsc.VectorSubcoreMesh(core_axis_name="c", subcore_axis_name="s")

    @functools.partial(
        pl.kernel,
        mesh=mesh,
        compiler_params=pltpu.CompilerParams(use_tc_tiling_on_sc=False),
        out_type=jax.ShapeDtypeStruct((total_rows, EMBED_DIM), jnp.float32),
        scratch_types=[
            pltpu.VMEM((cpw, CHUNK), jnp.int32),
            pltpu.VMEM((CHUNK, EMBED_DIM), jnp.float32),
            pltpu.SemaphoreType.DMA,
        ],
    )
    def gather_kernel(idx_hbm, table_hbm, out_hbm, idx_v, rows_v, gsem):
        wid = lax.axis_index("s") * 2 + lax.axis_index("c")
        cbase = wid * cpw
        pltpu.sync_copy(idx_hbm.at[pl.ds(cbase, cpw)], idx_v)

        def group_body(g, _):
            pltpu.async_copy(
                table_hbm.at[idx_v.at[g]], rows_v, gsem).wait()
            pltpu.sync_copy(
                rows_v, out_hbm.at[pl.ds((cbase + g) * CHUNK, CHUNK)])
            return 0

        lax.fori_loop(0, cpw, group_body, 0)

    return gather_kernel


def kernel(x, embeddings):
    batch, n_fields = x.shape
    total = batch * n_fields
    idx2d = x.reshape(total).astype(jnp.int32).reshape(total // CHUNK, CHUNK)
    out = _build_gather(total)(idx2d, embeddings)
    return out.reshape(batch, n_fields, EMBED_DIM)
